# baseline (speedup 1.0000x reference)
import functools

import jax
import jax.numpy as jnp
from jax import lax
from jax.experimental import pallas as pl
from jax.experimental.pallas import tpu as pltpu
from jax.experimental.pallas import tpu_sc as plsc

B, S, D = 4, 2048, 4096
SB = 256
SC_SEQ = 256
TC_SEQ = S - SC_SEQ
NS = TC_SEQ // SB
N = NS * B
KI = 6
KO = 6

HALF = 2048
L = 16
NC = 2
C = 8
HR_B = S * (D // HALF)
SC_HR = SC_SEQ * (D // HALF)
HR_W = (B * SC_HR) // 32
NCH = HR_W // C


def _tc_body(feat, pos, out, fb, ob, pb, sem_i, sem_o, sem_p):
    def in_copy(j):
        s, b = divmod(j, B)
        return pltpu.make_async_copy(
            feat.at[b, pl.ds(s * SB, SB), :], fb.at[j % KI], sem_i.at[j % KI])

    def out_copy(j):
        s, b = divmod(j, B)
        return pltpu.make_async_copy(
            ob.at[j % KO], out.at[b, pl.ds(s * SB, SB), :], sem_o.at[j % KO])

    def pos_copy(s):
        return pltpu.make_async_copy(
            pos.at[pl.ds(s * SB, SB), :], pb.at[s % 2], sem_p.at[s % 2])

    pos_copy(0).start()
    for j in range(KI):
        in_copy(j).start()

    for j in range(N):
        s, b = divmod(j, B)
        if b == 0:
            pos_copy(s).wait()
            if s + 1 < NS:
                pos_copy(s + 1).start()
        in_copy(j).wait()
        if j >= KO:
            out_copy(j - KO).wait()
        ob[j % KO] = fb[j % KI] + pb[s % 2]
        if j + KI < N:
            in_copy(j + KI).start()
        out_copy(j).start()

    for j in range(N - KO, N):
        out_copy(j).wait()


def _tc_kernel(feat, pos):
    return pl.pallas_call(
        _tc_body,
        in_specs=[
            pl.BlockSpec(memory_space=pl.ANY),
            pl.BlockSpec(memory_space=pl.ANY),
        ],
        out_specs=pl.BlockSpec(memory_space=pl.ANY),
        out_shape=jax.ShapeDtypeStruct((B, S, D), feat.dtype),
        scratch_shapes=[
            pltpu.VMEM((KI, SB, D), jnp.float32),
            pltpu.VMEM((KO, SB, D), jnp.float32),
            pltpu.VMEM((2, SB, D), jnp.float32),
            pltpu.SemaphoreType.DMA((KI,)),
            pltpu.SemaphoreType.DMA((KO,)),
            pltpu.SemaphoreType.DMA((2,)),
        ],
    )(feat, pos)


def _sc_body(feat_hbm, pos_hbm, out_hbm, fb, ob, pb,
             si0, si1, so0, so1, sp0, sp1):
    wid = lax.axis_index("s") * NC + lax.axis_index("c")
    batch = wid // 8
    lane = wid % 8
    fbase = batch * HR_B + (HR_B - SC_HR) + lane * HR_W
    pbase = (HR_B - SC_HR) + lane * HR_W
    obase = batch * SC_HR + lane * HR_W
    sems_i = (si0, si1)
    sems_o = (so0, so1)
    sems_p = (sp0, sp1)

    def in_copies(k):
        sl = k % 2
        return (
            pltpu.make_async_copy(
                feat_hbm.at[pl.ds(fbase + k * C, C)], fb.at[sl], sems_i[sl]),
            pltpu.make_async_copy(
                pos_hbm.at[pl.ds(pbase + k * C, C)], pb.at[sl], sems_p[sl]),
        )

    def out_copy(k):
        sl = k % 2
        return pltpu.make_async_copy(
            ob.at[sl], out_hbm.at[pl.ds(obase + k * C, C)], sems_o[sl])

    for cp in in_copies(0):
        cp.start()
    for k in range(NCH):
        sl = k % 2
        if k + 1 < NCH:
            for cp in in_copies(k + 1):
                cp.start()
        for cp in in_copies(k):
            cp.wait()
        if k >= 2:
            out_copy(k - 2).wait()
        for r in range(C):
            def add_vec(j, c, _sl=sl, _r=r):
                v = pl.ds(j * L, L)
                ob[_sl, _r, v] = fb[_sl, _r, v] + pb[_sl, _r, v]
                return c
            lax.fori_loop(0, HALF // L, add_vec, 0, unroll=8)
        out_copy(k).start()
    for k in range(NCH - 2, NCH):
        out_copy(k).wait()


def _sc_kernel(feat2, pos2):
    mesh = plsc.VectorSubcoreMesh(core_axis_name="c", subcore_axis_name="s")
    run = functools.partial(
        pl.kernel,
        mesh=mesh,
        out_type=jax.ShapeDtypeStruct((B * SC_HR, HALF), jnp.float32),
        scratch_types=[
            pltpu.VMEM((2, C, HALF), jnp.float32),
            pltpu.VMEM((2, C, HALF), jnp.float32),
            pltpu.VMEM((2, C, HALF), jnp.float32),
            pltpu.SemaphoreType.DMA,
            pltpu.SemaphoreType.DMA,
            pltpu.SemaphoreType.DMA,
            pltpu.SemaphoreType.DMA,
            pltpu.SemaphoreType.DMA,
            pltpu.SemaphoreType.DMA,
        ],
    )(_sc_body)
    return run(feat2, pos2)


def kernel(feat, pos_emb_weight):
    pos = pos_emb_weight[:S]
    feat2 = feat.reshape(B * HR_B, HALF)
    pos2 = pos.reshape(HR_B, HALF)
    sc_out = _sc_kernel(feat2, pos2).reshape(B, SC_SEQ, D)
    tc_out = _tc_kernel(feat, pos)
    return lax.dynamic_update_slice(tc_out, sc_out, (0, TC_SEQ, 0))


# speedup vs baseline: 2.4693x; 2.4693x over previous
import functools

import jax
import jax.numpy as jnp
from jax import lax
from jax.experimental import pallas as pl
from jax.experimental.pallas import tpu as pltpu
from jax.experimental.pallas import tpu_sc as plsc

B, S, D = 4, 2048, 4096
SB = 256
SC_SEQ = 256
TC_SEQ = S - SC_SEQ
NS = TC_SEQ // SB
N = NS * B
KI = 6
KO = 6

HALF = 2048
L = 16
NC = 2
C = 8
HR_B = S * (D // HALF)
SC_HR = SC_SEQ * (D // HALF)
HR_W = (B * SC_HR) // 32
NCH = HR_W // C


def _tc_body(feat, pos, out, fb, ob, pb, sem_i, sem_o, sem_p):
    def in_copy(j):
        s, b = divmod(j, B)
        return pltpu.make_async_copy(
            feat.at[b, pl.ds(s * SB, SB), :], fb.at[j % KI], sem_i.at[j % KI])

    def out_copy(j):
        s, b = divmod(j, B)
        return pltpu.make_async_copy(
            ob.at[j % KO], out.at[b, pl.ds(s * SB, SB), :], sem_o.at[j % KO])

    def pos_copy(s):
        return pltpu.make_async_copy(
            pos.at[pl.ds(s * SB, SB), :], pb.at[s % 2], sem_p.at[s % 2])

    pos_copy(0).start()
    for j in range(KI):
        in_copy(j).start()

    for j in range(N):
        s, b = divmod(j, B)
        if b == 0:
            pos_copy(s).wait()
            if s + 1 < NS:
                pos_copy(s + 1).start()
        in_copy(j).wait()
        if j >= KO:
            out_copy(j - KO).wait()
        ob[j % KO] = fb[j % KI] + pb[s % 2]
        if j + KI < N:
            in_copy(j + KI).start()
        out_copy(j).start()

    for j in range(N - KO, N):
        out_copy(j).wait()


def _tc_kernel(feat, pos):
    return pl.pallas_call(
        _tc_body,
        in_specs=[
            pl.BlockSpec(memory_space=pl.ANY),
            pl.BlockSpec(memory_space=pl.ANY),
        ],
        out_specs=pl.BlockSpec(memory_space=pl.ANY),
        out_shape=jax.ShapeDtypeStruct((B, S, D), feat.dtype),
        scratch_shapes=[
            pltpu.VMEM((KI, SB, D), jnp.float32),
            pltpu.VMEM((KO, SB, D), jnp.float32),
            pltpu.VMEM((2, SB, D), jnp.float32),
            pltpu.SemaphoreType.DMA((KI,)),
            pltpu.SemaphoreType.DMA((KO,)),
            pltpu.SemaphoreType.DMA((2,)),
        ],
    )(feat, pos)


def _sc_body(feat_hbm, pos_hbm, out_hbm, fb, ob, pb,
             si0, si1, so0, so1, sp0, sp1):
    wid = lax.axis_index("s") * NC + lax.axis_index("c")
    batch = wid // 8
    lane = wid % 8
    sq_w = SC_SEQ // 8
    cs = C // 2
    fbase = TC_SEQ + lane * sq_w
    obase = lane * sq_w
    sems_i = (si0, si1)
    sems_o = (so0, so1)
    sems_p = (sp0, sp1)

    def in_copies(k):
        sl = k % 2
        return (
            pltpu.make_async_copy(
                feat_hbm.at[batch, pl.ds(fbase + k * cs, cs), :],
                fb.at[sl], sems_i[sl]),
            pltpu.make_async_copy(
                pos_hbm.at[pl.ds(fbase + k * cs, cs), :], pb.at[sl],
                sems_p[sl]),
        )

    def out_copy(k):
        sl = k % 2
        return pltpu.make_async_copy(
            ob.at[sl], out_hbm.at[batch, pl.ds(obase + k * cs, cs), :],
            sems_o[sl])

    for cp in in_copies(0):
        cp.start()
    for k in range(NCH):
        sl = k % 2
        if k + 1 < NCH:
            for cp in in_copies(k + 1):
                cp.start()
        for cp in in_copies(k):
            cp.wait()
        if k >= 2:
            out_copy(k - 2).wait()
        for r in range(C // 2):
            def add_vec(j, c, _sl=sl, _r=r):
                v = pl.ds(j * L, L)
                ob[_sl, _r, v] = fb[_sl, _r, v] + pb[_sl, _r, v]
                return c
            lax.fori_loop(0, D // L, add_vec, 0, unroll=8)
        out_copy(k).start()
    for k in range(NCH - 2, NCH):
        out_copy(k).wait()


def _sc_kernel(feat2, pos2):
    mesh = plsc.VectorSubcoreMesh(core_axis_name="c", subcore_axis_name="s")
    run = functools.partial(
        pl.kernel,
        mesh=mesh,
        out_type=jax.ShapeDtypeStruct((B, SC_SEQ, D), jnp.float32),
        scratch_types=[
            pltpu.VMEM((2, C // 2, D), jnp.float32),
            pltpu.VMEM((2, C // 2, D), jnp.float32),
            pltpu.VMEM((2, C // 2, D), jnp.float32),
            pltpu.SemaphoreType.DMA,
            pltpu.SemaphoreType.DMA,
            pltpu.SemaphoreType.DMA,
            pltpu.SemaphoreType.DMA,
            pltpu.SemaphoreType.DMA,
            pltpu.SemaphoreType.DMA,
        ],
    )(_sc_body)
    return run(feat2, pos2)


def kernel(feat, pos_emb_weight):
    pos = pos_emb_weight[:S]
    sc_out = _sc_kernel(feat, pos)
    tc_out = _tc_kernel(feat, pos)
    return lax.dynamic_update_slice(tc_out, sc_out, (0, TC_SEQ, 0))


# speedup vs baseline: 3.3871x; 1.3717x over previous
import jax
import jax.numpy as jnp
from jax.experimental import pallas as pl
from jax.experimental.pallas import tpu as pltpu

B, S, D = 4, 2048, 4096
SB = 256
NS = S // SB
N = NS * B
KI = 6
KO = 6


def _body(feat, pos, out, fb, ob, pb, sem_i, sem_o, sem_p):
    def in_copy(j):
        s, b = divmod(j, B)
        return pltpu.make_async_copy(
            feat.at[b, pl.ds(s * SB, SB), :], fb.at[j % KI], sem_i.at[j % KI])

    def out_copy(j):
        s, b = divmod(j, B)
        return pltpu.make_async_copy(
            ob.at[j % KO], out.at[b, pl.ds(s * SB, SB), :], sem_o.at[j % KO])

    def pos_copy(s):
        return pltpu.make_async_copy(
            pos.at[pl.ds(s * SB, SB), :], pb.at[s % 2], sem_p.at[s % 2])

    pos_copy(0).start()
    for j in range(KI):
        in_copy(j).start()

    for j in range(N):
        s, b = divmod(j, B)
        if b == 0:
            pos_copy(s).wait()
            if s + 1 < NS:
                pos_copy(s + 1).start()
        in_copy(j).wait()
        if j >= KO:
            out_copy(j - KO).wait()
        ob[j % KO] = fb[j % KI] + pb[s % 2]
        if j + KI < N:
            in_copy(j + KI).start()
        out_copy(j).start()

    for j in range(N - KO, N):
        out_copy(j).wait()


def kernel(feat, pos_emb_weight):
    pos = pos_emb_weight[:S]
    return pl.pallas_call(
        _body,
        in_specs=[
            pl.BlockSpec(memory_space=pl.ANY),
            pl.BlockSpec(memory_space=pl.ANY),
        ],
        out_specs=pl.BlockSpec(memory_space=pl.ANY),
        out_shape=jax.ShapeDtypeStruct((B, S, D), feat.dtype),
        scratch_shapes=[
            pltpu.VMEM((KI, SB, D), jnp.float32),
            pltpu.VMEM((KO, SB, D), jnp.float32),
            pltpu.VMEM((2, SB, D), jnp.float32),
            pltpu.SemaphoreType.DMA((KI,)),
            pltpu.SemaphoreType.DMA((KO,)),
            pltpu.SemaphoreType.DMA((2,)),
        ],
    )(feat, pos)
